# Initial kernel scaffold; baseline (speedup 1.0000x reference)
import jax
import jax.numpy as jnp
from jax.experimental import pallas as pl


def kernel(feature_list_byAgentIdx, edge_index, W_pad, b_pad, W1, b1, W2, b2):
    raise NotImplementedError("write your pallas kernel here")


# speedup vs baseline: 10.7426x; 10.7426x over previous
import numpy as np
import jax
import jax.numpy as jnp
from jax.experimental import pallas as pl

_RAW_DIMS = np.array(
    [58, 58, 58, 82, 82, 82, 82, 58, 58, 58, 74, 58, 58, 58, 58, 58, 58, 74]
)
_N = 18
_P = 256
_F = 82
_FP = 128
_E = 128
_BB = 128

_MASK = (np.arange(_FP)[None, :] < _RAW_DIMS[:, None]).astype(np.float32)


def _gcn_kernel(ei_ref, mask_ref, feats_ref, wpad_ref, bpad_ref,
                w1_ref, b1_ref, w2_ref, b2_ref, out_ref):
    f32 = jnp.float32
    iota_n = jax.lax.broadcasted_iota(jnp.int32, (_N, _E), 0)
    src_oh = (ei_ref[0:1, :] == iota_n).astype(f32)
    dst_oh = (ei_ref[1:2, :] == iota_n).astype(f32)
    a_cnt = jax.lax.dot_general(dst_oh, src_oh, (((1,), (1,)), ((), ())),
                                preferred_element_type=f32)
    eye = (jax.lax.broadcasted_iota(jnp.int32, (_N, _N), 0)
           == jax.lax.broadcasted_iota(jnp.int32, (_N, _N), 1)).astype(f32)
    deg = jnp.sum(dst_oh, axis=1, keepdims=True) + 1.0
    dinv = jax.lax.rsqrt(deg)
    a_hat = (a_cnt + eye) * dinv * jnp.transpose(dinv)

    xm = feats_ref[...] * mask_ref[...][:, None, :]
    xs = jax.lax.dot_general(xm, wpad_ref[...],
                             (((2,), (1,)), ((0,), (0,))),
                             preferred_element_type=f32)
    xs = jnp.maximum(xs + bpad_ref[...][:, None, :], 0.0)

    h1 = jax.lax.dot_general(xs, w1_ref[...], (((2,), (0,)), ((), ())),
                             preferred_element_type=f32)
    p1 = jax.lax.dot_general(a_hat, h1, (((1,), (0,)), ((), ())),
                             preferred_element_type=f32)
    x1 = jnp.maximum(p1 + b1_ref[...][None, :, :], 0.0)

    h2 = jax.lax.dot_general(x1 + xs, w2_ref[...], (((2,), (0,)), ((), ())),
                             preferred_element_type=f32)
    p2 = jax.lax.dot_general(a_hat, h2, (((1,), (0,)), ((), ())),
                             preferred_element_type=f32)
    x2 = jnp.maximum(p2 + b2_ref[...][None, :, :], 0.0)

    out_ref[...] = jnp.swapaxes(x2 + x1, 0, 1)


def kernel(feature_list_byAgentIdx, edge_index, W_pad, b_pad, W1, b1, W2, b2):
    B = feature_list_byAgentIdx.shape[1]
    feats_p = jnp.pad(feature_list_byAgentIdx, ((0, 0), (0, 0), (0, _FP - _F)))
    wpad_p = jnp.pad(W_pad, ((0, 0), (0, _FP - _F), (0, 0)))
    mask = jnp.asarray(_MASK)
    grid = B // _BB

    return pl.pallas_call(
        _gcn_kernel,
        grid=(grid,),
        in_specs=[
            pl.BlockSpec((2, _E), lambda i: (0, 0)),
            pl.BlockSpec((_N, _FP), lambda i: (0, 0)),
            pl.BlockSpec((_N, _BB, _FP), lambda i: (0, i, 0)),
            pl.BlockSpec((_N, _FP, _P), lambda i: (0, 0, 0)),
            pl.BlockSpec((_N, _P), lambda i: (0, 0)),
            pl.BlockSpec((_P, _P), lambda i: (0, 0)),
            pl.BlockSpec((1, _P), lambda i: (0, 0)),
            pl.BlockSpec((_P, _P), lambda i: (0, 0)),
            pl.BlockSpec((1, _P), lambda i: (0, 0)),
        ],
        out_specs=pl.BlockSpec((_BB, _N, _P), lambda i: (i, 0, 0)),
        out_shape=jax.ShapeDtypeStruct((B, _N, _P), jnp.float32),
    )(edge_index, mask, feats_p, wpad_p, b_pad,
      W1, b1.reshape(1, _P), W2, b2.reshape(1, _P))


# speedup vs baseline: 13.2930x; 1.2374x over previous
import numpy as np
import jax
import jax.numpy as jnp
from jax.experimental import pallas as pl

_RAW_DIMS = np.array(
    [58, 58, 58, 82, 82, 82, 82, 58, 58, 58, 74, 58, 58, 58, 58, 58, 58, 74]
)
_N = 18
_P = 256
_F = 82
_E = 128
_BB = 128

_MASK = (np.arange(_F)[None, :] < _RAW_DIMS[:, None]).astype(np.float32)


def _gcn_kernel(ei_ref, mask_ref, feats_ref, wpad_ref, bpad_ref,
                w1_ref, b1_ref, w2_ref, b2_ref, out_ref):
    f32 = jnp.float32
    bf16 = jnp.bfloat16
    iota_n = jax.lax.broadcasted_iota(jnp.int32, (_N, _E), 0)
    src_oh = (ei_ref[0:1, :] == iota_n).astype(f32)
    dst_oh = (ei_ref[1:2, :] == iota_n).astype(f32)
    a_cnt = jax.lax.dot_general(dst_oh, src_oh, (((1,), (1,)), ((), ())),
                                preferred_element_type=f32)
    eye = (jax.lax.broadcasted_iota(jnp.int32, (_N, _N), 0)
           == jax.lax.broadcasted_iota(jnp.int32, (_N, _N), 1)).astype(f32)
    deg = jnp.sum(dst_oh, axis=1, keepdims=True) + 1.0
    dinv = jax.lax.rsqrt(deg)
    a_hat = ((a_cnt + eye) * dinv * jnp.transpose(dinv)).astype(bf16)

    xm = (feats_ref[...] * mask_ref[...][:, None, :]).astype(bf16)
    xs = jax.lax.dot_general(xm, wpad_ref[...],
                             (((2,), (1,)), ((0,), (0,))),
                             preferred_element_type=f32)
    xs = jnp.maximum(xs + bpad_ref[...][:, None, :], 0.0)

    h1 = jax.lax.dot_general(xs.astype(bf16), w1_ref[...],
                             (((2,), (0,)), ((), ())),
                             preferred_element_type=f32)
    p1 = jax.lax.dot_general(a_hat, h1.astype(bf16), (((1,), (0,)), ((), ())),
                             preferred_element_type=f32)
    x1 = jnp.maximum(p1 + b1_ref[...][None, :, :], 0.0)

    h2 = jax.lax.dot_general((x1 + xs).astype(bf16), w2_ref[...],
                             (((2,), (0,)), ((), ())),
                             preferred_element_type=f32)
    p2 = jax.lax.dot_general(a_hat, h2.astype(bf16), (((1,), (0,)), ((), ())),
                             preferred_element_type=f32)
    x2 = jnp.maximum(p2 + b2_ref[...][None, :, :], 0.0)

    x_out = x2 + x1
    for n in range(_N):
        out_ref[:, n, :] = x_out[n]


def kernel(feature_list_byAgentIdx, edge_index, W_pad, b_pad, W1, b1, W2, b2):
    B = feature_list_byAgentIdx.shape[1]
    mask = jnp.asarray(_MASK)
    grid = B // _BB

    return pl.pallas_call(
        _gcn_kernel,
        grid=(grid,),
        in_specs=[
            pl.BlockSpec((2, _E), lambda i: (0, 0)),
            pl.BlockSpec((_N, _F), lambda i: (0, 0)),
            pl.BlockSpec((_N, _BB, _F), lambda i: (0, i, 0)),
            pl.BlockSpec((_N, _F, _P), lambda i: (0, 0, 0)),
            pl.BlockSpec((_N, _P), lambda i: (0, 0)),
            pl.BlockSpec((_P, _P), lambda i: (0, 0)),
            pl.BlockSpec((1, _P), lambda i: (0, 0)),
            pl.BlockSpec((_P, _P), lambda i: (0, 0)),
            pl.BlockSpec((1, _P), lambda i: (0, 0)),
        ],
        out_specs=pl.BlockSpec((_BB, _N, _P), lambda i: (i, 0, 0)),
        out_shape=jax.ShapeDtypeStruct((B, _N, _P), jnp.float32),
    )(edge_index, mask, feature_list_byAgentIdx,
      W_pad.astype(jnp.bfloat16), b_pad,
      W1.astype(jnp.bfloat16), b1.reshape(1, _P),
      W2.astype(jnp.bfloat16), b2.reshape(1, _P))


# speedup vs baseline: 14.4222x; 1.0849x over previous
import numpy as np
import jax
import jax.numpy as jnp
from jax.experimental import pallas as pl

_RAW_DIMS = np.array(
    [58, 58, 58, 82, 82, 82, 82, 58, 58, 58, 74, 58, 58, 58, 58, 58, 58, 74]
)
_N = 18
_P = 256
_F = 82
_E = 128
_BB = 256

_MASK = (np.arange(_F)[None, :] < _RAW_DIMS[:, None]).astype(np.float32)


def _gcn_kernel(ei_ref, mask_ref, feats_ref, wpad_ref, bpad_ref,
                w1_ref, b1_ref, w2_ref, b2_ref, out_ref):
    f32 = jnp.float32
    bf16 = jnp.bfloat16
    iota_n = jax.lax.broadcasted_iota(jnp.int32, (_N, _E), 0)
    src_oh = (ei_ref[0:1, :] == iota_n).astype(f32)
    dst_oh = (ei_ref[1:2, :] == iota_n).astype(f32)
    a_cnt = jax.lax.dot_general(dst_oh, src_oh, (((1,), (1,)), ((), ())),
                                preferred_element_type=f32)
    eye = (jax.lax.broadcasted_iota(jnp.int32, (_N, _N), 0)
           == jax.lax.broadcasted_iota(jnp.int32, (_N, _N), 1)).astype(f32)
    deg = jnp.sum(dst_oh, axis=1, keepdims=True) + 1.0
    dinv = jax.lax.rsqrt(deg)
    a_hat = (a_cnt + eye) * dinv * jnp.transpose(dinv)

    xm = feats_ref[...] * mask_ref[...][:, None, :]
    xs = jax.lax.dot_general(xm, wpad_ref[...],
                             (((2,), (1,)), ((0,), (0,))),
                             preferred_element_type=f32)
    xs = jnp.maximum(xs + bpad_ref[...][:, None, :], 0.0)

    h1 = jax.lax.dot_general(xs, w1_ref[...],
                             (((2,), (0,)), ((), ())),
                             preferred_element_type=f32)
    p1 = jax.lax.dot_general(a_hat, h1, (((1,), (0,)), ((), ())),
                             preferred_element_type=f32)
    x1 = jnp.maximum(p1 + b1_ref[...][None, :, :], 0.0)

    h2 = jax.lax.dot_general(x1 + xs, w2_ref[...],
                             (((2,), (0,)), ((), ())),
                             preferred_element_type=f32)
    p2 = jax.lax.dot_general(a_hat, h2, (((1,), (0,)), ((), ())),
                             preferred_element_type=f32)
    x2 = jnp.maximum(p2 + b2_ref[...][None, :, :], 0.0)

    x_out = x2 + x1
    for n in range(_N):
        out_ref[:, n, :] = x_out[n]


def kernel(feature_list_byAgentIdx, edge_index, W_pad, b_pad, W1, b1, W2, b2):
    B = feature_list_byAgentIdx.shape[1]
    mask = jnp.asarray(_MASK)
    grid = B // _BB

    return pl.pallas_call(
        _gcn_kernel,
        grid=(grid,),
        in_specs=[
            pl.BlockSpec((2, _E), lambda i: (0, 0)),
            pl.BlockSpec((_N, _F), lambda i: (0, 0)),
            pl.BlockSpec((_N, _BB, _F), lambda i: (0, i, 0)),
            pl.BlockSpec((_N, _F, _P), lambda i: (0, 0, 0)),
            pl.BlockSpec((_N, _P), lambda i: (0, 0)),
            pl.BlockSpec((_P, _P), lambda i: (0, 0)),
            pl.BlockSpec((1, _P), lambda i: (0, 0)),
            pl.BlockSpec((_P, _P), lambda i: (0, 0)),
            pl.BlockSpec((1, _P), lambda i: (0, 0)),
        ],
        out_specs=pl.BlockSpec((_BB, _N, _P), lambda i: (i, 0, 0)),
        out_shape=jax.ShapeDtypeStruct((B, _N, _P), jnp.float32),
    )(edge_index, mask, feature_list_byAgentIdx,
      W_pad, b_pad,
      W1, b1.reshape(1, _P),
      W2, b2.reshape(1, _P))
